# Initial kernel scaffold; baseline (speedup 1.0000x reference)
import jax
import jax.numpy as jnp
from jax.experimental import pallas as pl


def kernel(x, edge_index, edge_attr, freqband_order, Wn, bn, We, be, layers, Wout, bout):
    raise NotImplementedError("write your pallas kernel here")


# speedup vs baseline: 4.0723x; 4.0723x over previous
import jax
import jax.numpy as jnp
from jax import lax
from jax.experimental import pallas as pl
from jax.experimental.pallas import tpu as pltpu
from jax.experimental.pallas import tpu_sc as plsc

N = 10000
E = 320000
H = 128
NSEG = 72

NUM_TILES = 32
CHUNK = 80
EPT = E // NUM_TILES
HALF = 5120
TRASH = HALF
ACC_ROWS = HALF + 8
CAP = EPT + 2 * CHUNK
NCAP = CAP // CHUNK
RPS = HALF // 16


def _route_kernel(src_hbm, dst_hbm, ea_hbm,
                  srcP_hbm, dstP_hbm, eaP_hbm, cnt_hbm,
                  sin, din, ain, sP, dP, aP, cnt_v, ptrA_v, ptrB_v, sem):
  c = lax.axis_index("c")
  s = lax.axis_index("s")
  wid = c * 16 + s

  pltpu.sync_copy(src_hbm.at[wid], sin)
  pltpu.sync_copy(dst_hbm.at[wid], din)
  pltpu.sync_copy(ea_hbm.at[wid], ain)

  lane = lax.iota(jnp.int32, 16)
  z16 = jnp.zeros((16,), jnp.int32)
  ptrA_v[...] = z16
  ptrB_v[...] = z16

  @pl.loop(0, EPT // 16)
  def _(i):
    ptrA = ptrA_v[...]
    ptrB = ptrB_v[...]
    sv = sin[pl.ds(16 * i, 16)]
    dv = din[pl.ds(16 * i, 16)]
    av = ain[pl.ds(16 * i, 16)]
    a = dv < HALF
    ai = a.astype(jnp.int32)
    incA = plsc.cumsum(ai)
    exA = incA - ai
    exB = lane - exA
    pos = jnp.where(a, ptrA + exA, (CAP - 1) - ptrB - exB)
    dval = jnp.where(a, dv, dv - HALF)
    plsc.store_scatter(sP, [pos], sv)
    plsc.store_scatter(dP, [pos], dval)
    plsc.store_scatter(aP, [pos], av)
    nA = plsc.all_reduce_population_count(a)
    ptrA_v[...] = ptrA + nA
    ptrB_v[...] = ptrB + (16 - nA)

  pav = ptrA_v[...]
  pbv = ptrB_v[...]
  ptrA = pav[0]
  ptrB = pbv[0]
  zi = jnp.zeros((16,), jnp.int32)
  zf = jnp.zeros((16,), jnp.float32)
  ti = jnp.full((16,), TRASH, jnp.int32)
  bbase = CAP - ptrB - CHUNK
  for t in range(CHUNK // 16):
    sP[pl.ds(ptrA + 16 * t, 16)] = zi
    dP[pl.ds(ptrA + 16 * t, 16)] = ti
    aP[pl.ds(ptrA + 16 * t, 16)] = zf
    sP[pl.ds(bbase + 16 * t, 16)] = zi
    dP[pl.ds(bbase + 16 * t, 16)] = ti
    aP[pl.ds(bbase + 16 * t, 16)] = zf

  chunksA = (ptrA + CHUNK - 1) // CHUNK
  chunksB = (ptrB + CHUNK - 1) // CHUNK
  cnt_v[...] = jnp.where(lane == 0, chunksA,
                         jnp.where(lane == 1, chunksB, 0))
  pltpu.sync_copy(cnt_v, cnt_hbm.at[wid])
  pltpu.sync_copy(sP, srcP_hbm.at[wid])
  pltpu.sync_copy(dP, dstP_hbm.at[wid])
  pltpu.sync_copy(aP, eaP_hbm.at[wid])


def _route_edges(src1, dst1, ea1):
  mesh = plsc.VectorSubcoreMesh(core_axis_name="c", subcore_axis_name="s")
  f = pl.kernel(
      _route_kernel,
      out_type=[jax.ShapeDtypeStruct((NUM_TILES, CAP), jnp.int32),
                jax.ShapeDtypeStruct((NUM_TILES, CAP), jnp.int32),
                jax.ShapeDtypeStruct((NUM_TILES, CAP), jnp.float32),
                jax.ShapeDtypeStruct((NUM_TILES, 16), jnp.int32)],
      mesh=mesh,
      compiler_params=pltpu.CompilerParams(needs_layout_passes=False),
      scratch_types=[
          pltpu.VMEM((EPT,), jnp.int32),
          pltpu.VMEM((EPT,), jnp.int32),
          pltpu.VMEM((EPT,), jnp.float32),
          pltpu.VMEM((CAP,), jnp.int32),
          pltpu.VMEM((CAP,), jnp.int32),
          pltpu.VMEM((CAP,), jnp.float32),
          pltpu.VMEM((16,), jnp.int32),
          pltpu.VMEM((16,), jnp.int32),
          pltpu.VMEM((16,), jnp.int32),
          pltpu.SemaphoreType.DMA,
      ],
  )
  return f(src1, dst1, ea1)


def _edge_kernel(hp_hbm, srcP_hbm, dstP_hbm, eaP_hbm, cnt_hbm, we_hbm,
                 out_hbm,
                 srcL, dstL, eaL, rows_v, zrow_v, we_v, cnt_v, sem, accum):
  c = lax.axis_index("c")
  s = lax.axis_index("s")
  wid = c * 16 + s

  pltpu.sync_copy(cnt_hbm.at[wid], cnt_v)
  cnts = cnt_v[...]
  pltpu.sync_copy(we_hbm, we_v)
  wvecs = [we_v[pl.ds(16 * v, 16)] for v in range(8)]
  pltpu.sync_copy(srcP_hbm.at[wid], srcL)
  pltpu.sync_copy(dstP_hbm.at[wid], dstL)
  pltpu.sync_copy(eaP_hbm.at[wid], eaL)

  zeros16 = jnp.zeros((16,), jnp.float32)
  @pl.loop(0, 160)
  def _(j):
    for v in range(8):
      zrow_v[j, pl.ds(16 * v, 16)] = zeros16

  pltpu.sync_copy(zrow_v, accum.at[pl.ds(s * RPS, 160)])
  pltpu.sync_copy(zrow_v, accum.at[pl.ds(s * RPS + 160, 160)])
  plsc.subcore_barrier()

  for passid, base in enumerate([0, HALF]):
    k_lo = jnp.where(passid == 0, 0, NCAP - cnts[1])
    k_hi = jnp.where(passid == 0, cnts[0], NCAP)

    @pl.loop(k_lo, k_hi)
    def _(k):
      pltpu.async_copy(hp_hbm.at[srcL.at[k]], rows_v, sem).wait()

      @pl.loop(0, CHUNK // 16)
      def _(g):
        eav = eaL[k, pl.ds(16 * g, 16)]
        for l in range(16):
          j = 16 * g + l
          eaj = eav[l]
          for v in range(8):
            hv = rows_v[j, pl.ds(16 * v, 16)]
            rows_v[j, pl.ds(16 * v, 16)] = jnp.maximum(hv + eaj * wvecs[v],
                                                       0.0)

      pltpu.sync_copy(rows_v, accum.at[dstL.at[k]], add=True)

    plsc.subcore_barrier()
    pltpu.sync_copy(accum.at[pl.ds(s * RPS, RPS)],
                    out_hbm.at[c, pl.ds(base + s * RPS, RPS)])
    if passid == 0:
      pltpu.sync_copy(zrow_v, accum.at[pl.ds(s * RPS, 160)])
      pltpu.sync_copy(zrow_v, accum.at[pl.ds(s * RPS + 160, 160)])
      plsc.subcore_barrier()


def _edge_aggregate(hp, srcP, dstP, eaP, cnts, we):
  mesh = plsc.VectorSubcoreMesh(core_axis_name="c", subcore_axis_name="s")
  f = pl.kernel(
      _edge_kernel,
      out_type=jax.ShapeDtypeStruct((2, 2 * HALF, H), jnp.float32),
      mesh=mesh,
      compiler_params=pltpu.CompilerParams(needs_layout_passes=False),
      scratch_types=[
          pltpu.VMEM((NCAP, CHUNK), jnp.int32),
          pltpu.VMEM((NCAP, CHUNK), jnp.int32),
          pltpu.VMEM((NCAP, CHUNK), jnp.float32),
          pltpu.VMEM((CHUNK, H), jnp.float32),
          pltpu.VMEM((160, H), jnp.float32),
          pltpu.VMEM((H,), jnp.float32),
          pltpu.VMEM((16,), jnp.int32),
          pltpu.SemaphoreType.DMA,
          pltpu.VMEM_SHARED((ACC_ROWS, H), jnp.float32),
      ],
  )
  return f(hp, srcP, dstP, eaP, cnts, we)


def _init_body(x_ref, wn_ref, bn_ref, be_ref, h_ref, hp_ref):
  h = x_ref[...] * wn_ref[...] + bn_ref[...]
  h_ref[...] = h
  hp_ref[...] = h + be_ref[...]


def _mlp_body(h_ref, a0_ref, a1_ref, w1_ref, b1_ref, w2_ref, b2_ref,
              eps_ref, be_ref, hout_ref, hpout_ref):
  hb = h_ref[...]
  out = hb + eps_ref[...] * hb + a0_ref[...] + a1_ref[...]
  t = jnp.maximum(
      jnp.dot(out, w1_ref[...], preferred_element_type=jnp.float32)
      + b1_ref[...], 0.0)
  o2 = jnp.dot(t, w2_ref[...], preferred_element_type=jnp.float32) + b2_ref[...]
  hn = hb + jnp.maximum(o2, 0.0)
  hout_ref[...] = hn
  hpout_ref[...] = hn + be_ref[...]


def _pool_body(h_ref, fb_ref, wout_ref, bout_ref, res_ref,
               pooled_acc, counts_acc):
  i = pl.program_id(0)

  @pl.when(i == 0)
  def _():
    pooled_acc[...] = jnp.zeros_like(pooled_acc)
    counts_acc[...] = jnp.zeros_like(counts_acc)

  hb = h_ref[...]
  seg = fb_ref[...]
  oh = (seg == lax.broadcasted_iota(jnp.int32, (seg.shape[0], NSEG), 1)
        ).astype(jnp.float32)
  dn = (((0,), (0,)), ((), ()))
  pooled_acc[...] += lax.dot_general(oh, hb, dn,
                                     preferred_element_type=jnp.float32)
  counts_acc[...] += lax.dot_general(oh, jnp.ones_like(hb), dn,
                                     preferred_element_type=jnp.float32)

  @pl.when(i == pl.num_programs(0) - 1)
  def _():
    cnt = counts_acc[...]
    pm = pooled_acc[...] / jnp.maximum(cnt, 1.0)
    out = (jnp.dot(pm, wout_ref[...], preferred_element_type=jnp.float32)
           + bout_ref[...])
    res_ref[...] = jnp.where(cnt > 0.0, out, 0.0)


BLK = 1000
GRID = N // BLK


def _row_spec():
  return pl.BlockSpec((BLK, H), lambda i: (i, 0))


def _full_spec(shape):
  return pl.BlockSpec(shape, lambda i: tuple(0 for _ in shape))


def kernel(x, edge_index, edge_attr, freqband_order, Wn, bn, We, be,
           layers, Wout, bout):
  src1 = edge_index[0].reshape(NUM_TILES, EPT)
  dst1 = edge_index[1].reshape(NUM_TILES, EPT)
  ea1 = edge_attr.reshape(NUM_TILES, EPT)
  we = We.reshape(H)
  bn2 = bn.reshape(1, H)
  be2 = be.reshape(1, H)
  bout2 = bout.reshape(1, H)
  fb2 = freqband_order.reshape(N, 1)

  epss = jnp.stack([lyr[0] for lyr in layers]).reshape(-1, 1, 1)
  ws1 = jnp.stack([lyr[1] for lyr in layers])
  bs1 = jnp.stack([lyr[2] for lyr in layers]).reshape(-1, 1, H)
  ws2 = jnp.stack([lyr[3] for lyr in layers])
  bs2 = jnp.stack([lyr[4] for lyr in layers]).reshape(-1, 1, H)
  nl = len(layers)

  srcP, dstP, eaP, cnts = _route_edges(src1, dst1, ea1)
  srcP = srcP.reshape(NUM_TILES, NCAP, CHUNK)
  dstP = dstP.reshape(NUM_TILES, NCAP, CHUNK)
  eaP = eaP.reshape(NUM_TILES, NCAP, CHUNK)

  h, hp = pl.pallas_call(
      _init_body,
      grid=(GRID,),
      in_specs=[pl.BlockSpec((BLK, 1), lambda i: (i, 0)),
                _full_spec((1, H)), _full_spec((1, H)), _full_spec((1, H))],
      out_specs=[_row_spec(), _row_spec()],
      out_shape=[jax.ShapeDtypeStruct((N, H), jnp.float32),
                 jax.ShapeDtypeStruct((N, H), jnp.float32)],
  )(x, Wn, bn2, be2)

  mlp_call = pl.pallas_call(
      _mlp_body,
      grid=(GRID,),
      in_specs=[_row_spec(), _row_spec(), _row_spec(),
                _full_spec((H, H)), _full_spec((1, H)),
                _full_spec((H, H)), _full_spec((1, H)),
                _full_spec((1, 1)), _full_spec((1, H))],
      out_specs=[_row_spec(), _row_spec()],
      out_shape=[jax.ShapeDtypeStruct((N, H), jnp.float32),
                 jax.ShapeDtypeStruct((N, H), jnp.float32)],
  )

  def body(li, carry):
    h, hp = carry
    aggr = _edge_aggregate(hp, srcP, dstP, eaP, cnts, we)
    w1 = lax.dynamic_index_in_dim(ws1, li, 0, keepdims=False)
    b1 = lax.dynamic_index_in_dim(bs1, li, 0, keepdims=False)
    w2 = lax.dynamic_index_in_dim(ws2, li, 0, keepdims=False)
    b2 = lax.dynamic_index_in_dim(bs2, li, 0, keepdims=False)
    eps = lax.dynamic_index_in_dim(epss, li, 0, keepdims=False)
    h, hp = mlp_call(h, aggr[0], aggr[1], w1, b1, w2, b2, eps, be2)
    return (h, hp)

  h, hp = lax.fori_loop(0, nl, body, (h, hp))

  res = pl.pallas_call(
      _pool_body,
      grid=(GRID,),
      in_specs=[_row_spec(),
                pl.BlockSpec((BLK, 1), lambda i: (i, 0)),
                _full_spec((H, H)), _full_spec((1, H))],
      out_specs=pl.BlockSpec((NSEG, H), lambda i: (0, 0)),
      out_shape=jax.ShapeDtypeStruct((NSEG, H), jnp.float32),
      scratch_shapes=[pltpu.VMEM((NSEG, H), jnp.float32),
                      pltpu.VMEM((NSEG, H), jnp.float32)],
  )(h, fb2, Wout, bout2)

  return res.reshape(8, 9, H)


# speedup vs baseline: 6.0460x; 1.4847x over previous
import jax
import jax.numpy as jnp
from jax import lax
from jax.experimental import pallas as pl
from jax.experimental.pallas import tpu as pltpu
from jax.experimental.pallas import tpu_sc as plsc

N = 10000
E = 320000
H = 128
NSEG = 72

NUM_TILES = 32
CHUNK = 80
EPT = E // NUM_TILES
HALF = 5120
TRASH = HALF
ACC_ROWS = HALF + 8
CAP = EPT + 2 * CHUNK
NCAP = CAP // CHUNK
RPS = HALF // 16


def _route_kernel(src_hbm, dst_hbm, ea_hbm,
                  srcP_hbm, dstP_hbm, eaP_hbm, cnt_hbm,
                  sin, din, ain, sP, dP, aP, cnt_v, ptrA_v, ptrB_v, sem):
  c = lax.axis_index("c")
  s = lax.axis_index("s")
  wid = c * 16 + s

  pltpu.sync_copy(src_hbm.at[wid], sin)
  pltpu.sync_copy(dst_hbm.at[wid], din)
  pltpu.sync_copy(ea_hbm.at[wid], ain)

  lane = lax.iota(jnp.int32, 16)
  z16 = jnp.zeros((16,), jnp.int32)
  ptrA_v[...] = z16
  ptrB_v[...] = z16

  @pl.loop(0, EPT // 16)
  def _(i):
    ptrA = ptrA_v[...]
    ptrB = ptrB_v[...]
    sv = sin[pl.ds(16 * i, 16)]
    dv = din[pl.ds(16 * i, 16)]
    av = ain[pl.ds(16 * i, 16)]
    a = dv < HALF
    ai = a.astype(jnp.int32)
    incA = plsc.cumsum(ai)
    exA = incA - ai
    exB = lane - exA
    pos = jnp.where(a, ptrA + exA, (CAP - 1) - ptrB - exB)
    dval = jnp.where(a, dv, dv - HALF)
    plsc.store_scatter(sP, [pos], sv)
    plsc.store_scatter(dP, [pos], dval)
    plsc.store_scatter(aP, [pos], av)
    nA = plsc.all_reduce_population_count(a)
    ptrA_v[...] = ptrA + nA
    ptrB_v[...] = ptrB + (16 - nA)

  pav = ptrA_v[...]
  pbv = ptrB_v[...]
  ptrA = pav[0]
  ptrB = pbv[0]
  zi = jnp.zeros((16,), jnp.int32)
  zf = jnp.zeros((16,), jnp.float32)
  ti = jnp.full((16,), TRASH, jnp.int32)
  bbase = CAP - ptrB - CHUNK
  for t in range(CHUNK // 16):
    sP[pl.ds(ptrA + 16 * t, 16)] = zi
    dP[pl.ds(ptrA + 16 * t, 16)] = ti
    aP[pl.ds(ptrA + 16 * t, 16)] = zf
    sP[pl.ds(bbase + 16 * t, 16)] = zi
    dP[pl.ds(bbase + 16 * t, 16)] = ti
    aP[pl.ds(bbase + 16 * t, 16)] = zf

  chunksA = (ptrA + CHUNK - 1) // CHUNK
  chunksB = (ptrB + CHUNK - 1) // CHUNK
  cnt_v[...] = jnp.where(lane == 0, chunksA,
                         jnp.where(lane == 1, chunksB, 0))
  pltpu.sync_copy(cnt_v, cnt_hbm.at[wid])
  pltpu.sync_copy(sP, srcP_hbm.at[wid])
  pltpu.sync_copy(dP, dstP_hbm.at[wid])
  pltpu.sync_copy(aP, eaP_hbm.at[wid])


def _route_edges(src1, dst1, ea1):
  mesh = plsc.VectorSubcoreMesh(core_axis_name="c", subcore_axis_name="s")
  f = pl.kernel(
      _route_kernel,
      out_type=[jax.ShapeDtypeStruct((NUM_TILES, CAP), jnp.int32),
                jax.ShapeDtypeStruct((NUM_TILES, CAP), jnp.int32),
                jax.ShapeDtypeStruct((NUM_TILES, CAP), jnp.float32),
                jax.ShapeDtypeStruct((NUM_TILES, 16), jnp.int32)],
      mesh=mesh,
      compiler_params=pltpu.CompilerParams(needs_layout_passes=False),
      scratch_types=[
          pltpu.VMEM((EPT,), jnp.int32),
          pltpu.VMEM((EPT,), jnp.int32),
          pltpu.VMEM((EPT,), jnp.float32),
          pltpu.VMEM((CAP,), jnp.int32),
          pltpu.VMEM((CAP,), jnp.int32),
          pltpu.VMEM((CAP,), jnp.float32),
          pltpu.VMEM((16,), jnp.int32),
          pltpu.VMEM((16,), jnp.int32),
          pltpu.VMEM((16,), jnp.int32),
          pltpu.SemaphoreType.DMA,
      ],
  )
  return f(src1, dst1, ea1)


def _edge_kernel(hp_hbm, srcP_hbm, dstP_hbm, eaP_hbm, cnt_hbm, we_hbm,
                 out_hbm,
                 srcL, dstL, eaL, rows_v, rows2_v, zrow_v, we_v, cnt_v, sem, accum):
  c = lax.axis_index("c")
  s = lax.axis_index("s")
  wid = c * 16 + s

  pltpu.sync_copy(cnt_hbm.at[wid], cnt_v)
  cnts = cnt_v[...]
  pltpu.sync_copy(we_hbm, we_v)
  wvecs = [we_v[pl.ds(16 * v, 16)] for v in range(8)]
  pltpu.sync_copy(srcP_hbm.at[wid], srcL)
  pltpu.sync_copy(dstP_hbm.at[wid], dstL)
  pltpu.sync_copy(eaP_hbm.at[wid], eaL)

  zeros16 = jnp.zeros((16,), jnp.float32)
  @pl.loop(0, 80)
  def _(j):
    for v in range(8):
      zrow_v[j, pl.ds(16 * v, 16)] = zeros16

  for r in range(4):
    pltpu.sync_copy(zrow_v, accum.at[pl.ds(s * RPS + 80 * r, 80)])
  plsc.subcore_barrier()

  def compute(buf, k):
    @pl.loop(0, CHUNK // 16)
    def _(g):
      eav = eaL[k, pl.ds(16 * g, 16)]
      for l in range(16):
        j = 16 * g + l
        eaj = eav[l]
        for v in range(8):
          hv = buf[j, pl.ds(16 * v, 16)]
          buf[j, pl.ds(16 * v, 16)] = jnp.maximum(hv + eaj * wvecs[v], 0.0)

    pltpu.sync_copy(buf, accum.at[dstL.at[k]], add=True)

  def fire(buf, k):
    pltpu.async_copy(hp_hbm.at[srcL.at[k]], buf, sem)

  def drain(buf):
    pltpu.make_async_copy(hp_hbm.at[srcL.at[0]], buf, sem).wait()

  for passid, base in enumerate([0, HALF]):
    k_lo = jnp.where(passid == 0, 0, NCAP - cnts[1])
    k_hi = jnp.where(passid == 0, cnts[0], NCAP)
    n = k_hi - k_lo

    @pl.when(n > 0)
    def _():
      fire(rows_v, k_lo)

    @pl.loop(0, (n + 1) // 2)
    def _(g):
      k0 = k_lo + 2 * g

      @pl.when(k0 + 1 < k_hi)
      def _():
        fire(rows2_v, k0 + 1)

      drain(rows_v)
      compute(rows_v, k0)

      @pl.when(k0 + 1 < k_hi)
      def _():
        @pl.when(k0 + 2 < k_hi)
        def _():
          fire(rows_v, k0 + 2)

        drain(rows2_v)
        compute(rows2_v, k0 + 1)

    plsc.subcore_barrier()
    pltpu.sync_copy(accum.at[pl.ds(s * RPS, RPS)],
                    out_hbm.at[c, pl.ds(base + s * RPS, RPS)])
    if passid == 0:
      for r in range(4):
        pltpu.sync_copy(zrow_v, accum.at[pl.ds(s * RPS + 80 * r, 80)])
      plsc.subcore_barrier()


def _edge_aggregate(hp, srcP, dstP, eaP, cnts, we):
  mesh = plsc.VectorSubcoreMesh(core_axis_name="c", subcore_axis_name="s")
  f = pl.kernel(
      _edge_kernel,
      out_type=jax.ShapeDtypeStruct((2, 2 * HALF, H), jnp.float32),
      mesh=mesh,
      compiler_params=pltpu.CompilerParams(needs_layout_passes=False),
      scratch_types=[
          pltpu.VMEM((NCAP, CHUNK), jnp.int32),
          pltpu.VMEM((NCAP, CHUNK), jnp.int32),
          pltpu.VMEM((NCAP, CHUNK), jnp.float32),
          pltpu.VMEM((CHUNK, H), jnp.float32),
          pltpu.VMEM((CHUNK, H), jnp.float32),
          pltpu.VMEM((80, H), jnp.float32),
          pltpu.VMEM((H,), jnp.float32),
          pltpu.VMEM((16,), jnp.int32),
          pltpu.SemaphoreType.DMA,
          pltpu.VMEM_SHARED((ACC_ROWS, H), jnp.float32),
      ],
  )
  return f(hp, srcP, dstP, eaP, cnts, we)


def _edge1_kernel(x_hbm, srcP_hbm, dstP_hbm, eaP_hbm, cnt_hbm,
                  wn_hbm, bn_hbm, we_hbm, out_hbm,
                  srcL, dstL, eaL, rows_v, zrow_v, x_v, wn_v, bn_v, we_v,
                  cnt_v, sem, accum):
  c = lax.axis_index("c")
  s = lax.axis_index("s")
  wid = c * 16 + s

  pltpu.sync_copy(cnt_hbm.at[wid], cnt_v)
  cnts = cnt_v[...]
  pltpu.sync_copy(wn_hbm, wn_v)
  pltpu.sync_copy(bn_hbm, bn_v)
  pltpu.sync_copy(we_hbm, we_v)
  pltpu.sync_copy(x_hbm, x_v)
  wnv = [wn_v[pl.ds(16 * v, 16)] for v in range(8)]
  bnv = [bn_v[pl.ds(16 * v, 16)] for v in range(8)]
  wev = [we_v[pl.ds(16 * v, 16)] for v in range(8)]
  pltpu.sync_copy(srcP_hbm.at[wid], srcL)
  pltpu.sync_copy(dstP_hbm.at[wid], dstL)
  pltpu.sync_copy(eaP_hbm.at[wid], eaL)

  zeros16 = jnp.zeros((16,), jnp.float32)
  @pl.loop(0, 80)
  def _(j):
    for v in range(8):
      zrow_v[j, pl.ds(16 * v, 16)] = zeros16

  for r in range(4):
    pltpu.sync_copy(zrow_v, accum.at[pl.ds(s * RPS + 80 * r, 80)])
  plsc.subcore_barrier()

  for passid, base in enumerate([0, HALF]):
    k_lo = jnp.where(passid == 0, 0, NCAP - cnts[1])
    k_hi = jnp.where(passid == 0, cnts[0], NCAP)

    @pl.loop(k_lo, k_hi)
    def _(k):
      @pl.loop(0, CHUNK // 16)
      def _(g):
        src16 = srcL[k, pl.ds(16 * g, 16)]
        xs16 = plsc.load_gather(x_v, [src16])
        eav = eaL[k, pl.ds(16 * g, 16)]
        for l in range(16):
          j = 16 * g + l
          xs = xs16[l]
          eaj = eav[l]
          for v in range(8):
            rows_v[j, pl.ds(16 * v, 16)] = jnp.maximum(
                xs * wnv[v] + (eaj * wev[v] + bnv[v]), 0.0)

      pltpu.sync_copy(rows_v, accum.at[dstL.at[k]], add=True)

    plsc.subcore_barrier()
    pltpu.sync_copy(accum.at[pl.ds(s * RPS, RPS)],
                    out_hbm.at[c, pl.ds(base + s * RPS, RPS)])
    if passid == 0:
      for r in range(4):
        pltpu.sync_copy(zrow_v, accum.at[pl.ds(s * RPS + 80 * r, 80)])
      plsc.subcore_barrier()


def _edge1_aggregate(x1, srcP, dstP, eaP, cnts, wn, bn, we):
  mesh = plsc.VectorSubcoreMesh(core_axis_name="c", subcore_axis_name="s")
  f = pl.kernel(
      _edge1_kernel,
      out_type=jax.ShapeDtypeStruct((2, 2 * HALF, H), jnp.float32),
      mesh=mesh,
      compiler_params=pltpu.CompilerParams(needs_layout_passes=False),
      scratch_types=[
          pltpu.VMEM((NCAP, CHUNK), jnp.int32),
          pltpu.VMEM((NCAP, CHUNK), jnp.int32),
          pltpu.VMEM((NCAP, CHUNK), jnp.float32),
          pltpu.VMEM((CHUNK, H), jnp.float32),
          pltpu.VMEM((80, H), jnp.float32),
          pltpu.VMEM((N,), jnp.float32),
          pltpu.VMEM((H,), jnp.float32),
          pltpu.VMEM((H,), jnp.float32),
          pltpu.VMEM((H,), jnp.float32),
          pltpu.VMEM((16,), jnp.int32),
          pltpu.SemaphoreType.DMA,
          pltpu.VMEM_SHARED((ACC_ROWS, H), jnp.float32),
      ],
  )
  return f(x1, srcP, dstP, eaP, cnts, wn, bn, we)


def _init_body(x_ref, wn_ref, bn_ref, be_ref, h_ref, hp_ref):
  h = x_ref[...] * wn_ref[...] + bn_ref[...]
  h_ref[...] = h
  hp_ref[...] = h + be_ref[...]


def _mlp_body(h_ref, a0_ref, a1_ref, w1_ref, b1_ref, w2_ref, b2_ref,
              eps_ref, be_ref, hout_ref, hpout_ref):
  hb = h_ref[...]
  out = hb + eps_ref[...] * hb + a0_ref[...] + a1_ref[...]
  t = jnp.maximum(
      jnp.dot(out, w1_ref[...], preferred_element_type=jnp.float32)
      + b1_ref[...], 0.0)
  o2 = jnp.dot(t, w2_ref[...], preferred_element_type=jnp.float32) + b2_ref[...]
  hn = hb + jnp.maximum(o2, 0.0)
  hout_ref[...] = hn
  hpout_ref[...] = hn + be_ref[...]


def _pool_body(h_ref, fb_ref, wout_ref, bout_ref, res_ref,
               pooled_acc, counts_acc):
  i = pl.program_id(0)

  @pl.when(i == 0)
  def _():
    pooled_acc[...] = jnp.zeros_like(pooled_acc)
    counts_acc[...] = jnp.zeros_like(counts_acc)

  hb = h_ref[...]
  seg = fb_ref[...]
  oh = (seg == lax.broadcasted_iota(jnp.int32, (seg.shape[0], NSEG), 1)
        ).astype(jnp.float32)
  dn = (((0,), (0,)), ((), ()))
  pooled_acc[...] += lax.dot_general(oh, hb, dn,
                                     preferred_element_type=jnp.float32)
  counts_acc[...] += lax.dot_general(oh, jnp.ones_like(hb), dn,
                                     preferred_element_type=jnp.float32)

  @pl.when(i == pl.num_programs(0) - 1)
  def _():
    cnt = counts_acc[...]
    pm = pooled_acc[...] / jnp.maximum(cnt, 1.0)
    out = (jnp.dot(pm, wout_ref[...], preferred_element_type=jnp.float32)
           + bout_ref[...])
    res_ref[...] = jnp.where(cnt > 0.0, out, 0.0)


BLK = 1000
GRID = N // BLK


def _row_spec():
  return pl.BlockSpec((BLK, H), lambda i: (i, 0))


def _full_spec(shape):
  return pl.BlockSpec(shape, lambda i: tuple(0 for _ in shape))


def kernel(x, edge_index, edge_attr, freqband_order, Wn, bn, We, be,
           layers, Wout, bout):
  src1 = edge_index[0].reshape(NUM_TILES, EPT)
  dst1 = edge_index[1].reshape(NUM_TILES, EPT)
  ea1 = edge_attr.reshape(NUM_TILES, EPT)
  we = We.reshape(H)
  bn2 = bn.reshape(1, H)
  be2 = be.reshape(1, H)
  bout2 = bout.reshape(1, H)
  fb2 = freqband_order.reshape(N, 1)

  epss = jnp.stack([lyr[0] for lyr in layers[1:]]).reshape(-1, 1, 1)
  ws1 = jnp.stack([lyr[1] for lyr in layers[1:]])
  bs1 = jnp.stack([lyr[2] for lyr in layers[1:]]).reshape(-1, 1, H)
  ws2 = jnp.stack([lyr[3] for lyr in layers[1:]])
  bs2 = jnp.stack([lyr[4] for lyr in layers[1:]]).reshape(-1, 1, H)
  nl = len(layers) - 1
  wn1 = Wn.reshape(H)
  bn1 = bn.reshape(H)
  x1 = x.reshape(N)

  srcP, dstP, eaP, cnts = _route_edges(src1, dst1, ea1)
  srcP = srcP.reshape(NUM_TILES, NCAP, CHUNK)
  dstP = dstP.reshape(NUM_TILES, NCAP, CHUNK)
  eaP = eaP.reshape(NUM_TILES, NCAP, CHUNK)

  h, hp = pl.pallas_call(
      _init_body,
      grid=(GRID,),
      in_specs=[pl.BlockSpec((BLK, 1), lambda i: (i, 0)),
                _full_spec((1, H)), _full_spec((1, H)), _full_spec((1, H))],
      out_specs=[_row_spec(), _row_spec()],
      out_shape=[jax.ShapeDtypeStruct((N, H), jnp.float32),
                 jax.ShapeDtypeStruct((N, H), jnp.float32)],
  )(x, Wn, bn2, be2)

  mlp_call = pl.pallas_call(
      _mlp_body,
      grid=(GRID,),
      in_specs=[_row_spec(), _row_spec(), _row_spec(),
                _full_spec((H, H)), _full_spec((1, H)),
                _full_spec((H, H)), _full_spec((1, H)),
                _full_spec((1, 1)), _full_spec((1, H))],
      out_specs=[_row_spec(), _row_spec()],
      out_shape=[jax.ShapeDtypeStruct((N, H), jnp.float32),
                 jax.ShapeDtypeStruct((N, H), jnp.float32)],
  )

  eps0, w10, b10, w20, b20 = layers[0]
  aggr = _edge1_aggregate(x1, srcP, dstP, eaP, cnts, wn1, bn1, we)
  h, hp = mlp_call(h, aggr[0], aggr[1], w10, b10.reshape(1, H),
                   w20, b20.reshape(1, H), eps0.reshape(1, 1), be2)

  def body(li, carry):
    h, hp = carry
    aggr = _edge_aggregate(hp, srcP, dstP, eaP, cnts, we)
    w1 = lax.dynamic_index_in_dim(ws1, li, 0, keepdims=False)
    b1 = lax.dynamic_index_in_dim(bs1, li, 0, keepdims=False)
    w2 = lax.dynamic_index_in_dim(ws2, li, 0, keepdims=False)
    b2 = lax.dynamic_index_in_dim(bs2, li, 0, keepdims=False)
    eps = lax.dynamic_index_in_dim(epss, li, 0, keepdims=False)
    h, hp = mlp_call(h, aggr[0], aggr[1], w1, b1, w2, b2, eps, be2)
    return (h, hp)

  h, hp = lax.fori_loop(0, nl, body, (h, hp))

  res = pl.pallas_call(
      _pool_body,
      grid=(GRID,),
      in_specs=[_row_spec(),
                pl.BlockSpec((BLK, 1), lambda i: (i, 0)),
                _full_spec((H, H)), _full_spec((1, H))],
      out_specs=pl.BlockSpec((NSEG, H), lambda i: (0, 0)),
      out_shape=jax.ShapeDtypeStruct((NSEG, H), jnp.float32),
      scratch_shapes=[pltpu.VMEM((NSEG, H), jnp.float32),
                      pltpu.VMEM((NSEG, H), jnp.float32)],
  )(h, fb2, Wout, bout2)

  return res.reshape(8, 9, H)
